# baseline (speedup 1.0000x reference)
import functools

import jax
import jax.numpy as jnp
from jax import lax
from jax.experimental import pallas as pl
from jax.experimental.pallas import tpu as pltpu
from jax.experimental.pallas import tpu_sc as plsc

B, L, D = 4096, 200, 128
NL = 10
LP = 208
CH = LP // 2
NC, NS, LANES = 2, 16, 16
NW = NC * NS
RPW = B // NW
NVR = D // LANES


NBUF = 4
NCH = RPW * 2


def _make_bag():
    mesh = plsc.VectorSubcoreMesh(core_axis_name="c", subcore_axis_name="s")

    @functools.partial(
        pl.kernel,
        mesh=mesh,
        out_type=jax.ShapeDtypeStruct((B, D), jnp.float32),
        scratch_types=[
            pltpu.VMEM((RPW * LP,), jnp.int32),
            pltpu.VMEM((NBUF, CH, D), jnp.float32),
            pltpu.VMEM((RPW, D), jnp.float32),
            pltpu.SemaphoreType.DMA,
            pltpu.SemaphoreType.DMA,
            pltpu.SemaphoreType.DMA,
            pltpu.SemaphoreType.DMA,
        ],
    )
    def bag(x_hbm, emb_hbm, out_hbm, idx_v, rows_v, z_v, s0, s1, s2, s3):
        sems = (s0, s1, s2, s3)
        wid = lax.axis_index("s") * NC + lax.axis_index("c")
        base = wid * RPW
        pltpu.sync_copy(
            x_hbm.at[pl.ds(pl.multiple_of(base * LP, 8), RPW * LP)], idx_v)

        def copy_desc(c, slot):
            off = pl.multiple_of(c * CH, 8)
            return pltpu.make_async_copy(
                emb_hbm.at[idx_v.at[pl.ds(off, CH)]], rows_v.at[slot],
                sems[slot])

        for k in range(NBUF - 1):
            copy_desc(k, k).start()

        def reduce_chunk(slot, accs):
            def red(r, accs):
                return tuple(
                    a + rows_v[slot, 2 * r, pl.ds(j * LANES, LANES)]
                    + rows_v[slot, 2 * r + 1, pl.ds(j * LANES, LANES)]
                    for j, a in enumerate(accs))
            return lax.fori_loop(0, CH // 2, red, accs)

        zero = tuple(jnp.zeros((LANES,), jnp.float32) for _ in range(NVR))

        def group_body(g, carry):
            for k in range(NBUF):
                c = 4 * g + k
                f = c + NBUF - 1
                fslot = (k + NBUF - 1) % NBUF

                @pl.when(f < NCH)
                def _():
                    copy_desc(f, fslot).start()

                copy_desc(c, k).wait()
                if k % 2 == 0:
                    accs = reduce_chunk(k, zero)
                else:
                    accs = reduce_chunk(k, accs)
                    row = 2 * g + k // 2
                    for j in range(NVR):
                        z_v[row, pl.ds(j * LANES, LANES)] = accs[j]
            return carry

        lax.fori_loop(0, NCH // NBUF, group_body, 0)
        pltpu.sync_copy(z_v, out_hbm.at[pl.ds(base, RPW)])

    return bag


_bag = _make_bag()


def _head_body(s_ref, x_ref, wp_ref, bp_ref, wf_ref, bf_ref, o_ref):
    cnt = jnp.sum((x_ref[...] != 0).astype(jnp.float32), axis=1, keepdims=True)
    z = s_ref[...] / jnp.maximum(cnt, 1.0)
    h = lax.dot_general(z, wp_ref[...], (((1,), (1,)), ((), ())),
                        preferred_element_type=jnp.float32)
    h = jnp.maximum(h + bp_ref[...], 0.0)
    o = lax.dot_general(h, wf_ref[...], (((1,), (1,)), ((), ())),
                        preferred_element_type=jnp.float32)
    o_ref[...] = o + bf_ref[...]


BT = 512


_head = pl.pallas_call(
    _head_body,
    grid=(B // BT,),
    in_specs=[
        pl.BlockSpec((BT, D), lambda i: (i, 0)),
        pl.BlockSpec((BT, L), lambda i: (i, 0)),
        pl.BlockSpec((D, D), lambda i: (0, 0)),
        pl.BlockSpec((1, D), lambda i: (0, 0)),
        pl.BlockSpec((NL, D), lambda i: (0, 0)),
        pl.BlockSpec((1, NL), lambda i: (0, 0)),
    ],
    out_specs=pl.BlockSpec((BT, NL), lambda i: (i, 0)),
    out_shape=jax.ShapeDtypeStruct((B, NL), jnp.float32),
)


def kernel(x, emb, Wp, bp, Wf, bf):
    x_pad = jnp.pad(x, ((0, 0), (0, LP - L))).reshape(B * LP)
    sums = _bag(x_pad, emb)
    return _head(sums, x, Wp, bp.reshape(1, D), Wf, bf.reshape(1, NL))
